# baseline (speedup 1.0000x reference)
import functools

import jax
import jax.numpy as jnp
from jax import lax
from jax.experimental import pallas as pl
from jax.experimental.pallas import tpu as pltpu
from jax.experimental.pallas import tpu_sc as plsc

NC = 2
NS = 16
NW = NC * NS
L = 16
BATCH = 16384
NF = 32
BPW = BATCH // NW


def _body(user_hbm, item_hbm, uf_hbm, if_hbm, out_hbm,
          uidx_v, iidx_v, urows_v, vrows_v, out_v, usem, vsem):
    wid = lax.axis_index("s") * NC + lax.axis_index("c")
    base = wid * BPW

    pltpu.sync_copy(user_hbm.at[pl.ds(base, BPW)], uidx_v)
    pltpu.sync_copy(item_hbm.at[pl.ds(base, BPW)], iidx_v)
    cu = pltpu.async_copy(uf_hbm.at[uidx_v], urows_v, usem)
    cv = pltpu.async_copy(if_hbm.at[iidx_v], vrows_v, vsem)
    cu.wait()
    cv.wait()

    lane = lax.iota(jnp.int32, L)

    def group(g, carry):
        def elem(j, acc):
            b = g * L + j
            u0 = urows_v[b, pl.ds(0, L)]
            u1 = urows_v[b, pl.ds(L, L)]
            v0 = vrows_v[b, pl.ds(0, L)]
            v1 = vrows_v[b, pl.ds(L, L)]
            c = u0 * v0 + u1 * v1
            s = jnp.sum(c)
            return jnp.where(lane == j, s, acc)

        acc = lax.fori_loop(0, L, elem, jnp.zeros((L,), jnp.float32))
        out_v[pl.ds(g * L, L)] = acc
        return carry

    lax.fori_loop(0, BPW // L, group, 0)
    pltpu.sync_copy(out_v, out_hbm.at[pl.ds(base, BPW)])


@jax.jit
def kernel(user, item, user_factors, item_factors):
    k = pl.kernel(
        _body,
        out_type=jax.ShapeDtypeStruct((BATCH,), jnp.float32),
        mesh=plsc.VectorSubcoreMesh(
            core_axis_name="c", subcore_axis_name="s",
            num_cores=NC, num_subcores=NS),
        compiler_params=pltpu.CompilerParams(
            needs_layout_passes=False, use_tc_tiling_on_sc=False),
        scratch_types=[
            pltpu.VMEM((BPW,), jnp.int32),
            pltpu.VMEM((BPW,), jnp.int32),
            pltpu.VMEM((BPW, NF), jnp.float32),
            pltpu.VMEM((BPW, NF), jnp.float32),
            pltpu.VMEM((BPW,), jnp.float32),
            pltpu.SemaphoreType.DMA,
            pltpu.SemaphoreType.DMA,
        ],
    )
    return k(user, item, user_factors, item_factors)


# speedup vs baseline: 1.5008x; 1.5008x over previous
import functools

import jax
import jax.numpy as jnp
from jax import lax
from jax.experimental import pallas as pl
from jax.experimental.pallas import tpu as pltpu
from jax.experimental.pallas import tpu_sc as plsc

NC = 2
NS = 16
NW = NC * NS
L = 16
BATCH = 16384
NF = 32
BPW = BATCH // NW
W = 256


def _body(user_hbm, item_hbm, uf_hbm, if_hbm, out_hbm,
          uidx_v, iidx_v, urows_v, vrows_v, out_v, usem, vsem):
    wid = lax.axis_index("s") * NC + lax.axis_index("c")
    base = wid * BPW

    pltpu.sync_copy(user_hbm.at[pl.ds(base, BPW)], uidx_v)
    pltpu.sync_copy(item_hbm.at[pl.ds(base, BPW)], iidx_v)

    lane = lax.iota(jnp.int32, L)

    def window(w, carry):
        w0 = w * W

        def fire(k, c):
            uv = uidx_v[pl.ds(w0 + k * L, L)]
            iv = iidx_v[pl.ds(w0 + k * L, L)]
            for j in range(L):
                pltpu.async_copy(
                    uf_hbm.at[pl.ds(uv[j], 1)],
                    urows_v.at[pl.ds(k * L + j, 1)], usem)
                pltpu.async_copy(
                    if_hbm.at[pl.ds(iv[j], 1)],
                    vrows_v.at[pl.ds(k * L + j, 1)], vsem)
            return c

        lax.fori_loop(0, W // L, fire, 0)

        def drain(i, c):
            pltpu.make_async_copy(
                uf_hbm.at[pl.ds(0, 1)], urows_v.at[pl.ds(0, 1)], usem).wait()
            pltpu.make_async_copy(
                if_hbm.at[pl.ds(0, 1)], vrows_v.at[pl.ds(0, 1)], vsem).wait()
            return c

        lax.fori_loop(0, W, drain, 0)

        def group(g, c):
            def elem(j, acc):
                b = g * L + j
                u0 = urows_v[b, pl.ds(0, L)]
                u1 = urows_v[b, pl.ds(L, L)]
                v0 = vrows_v[b, pl.ds(0, L)]
                v1 = vrows_v[b, pl.ds(L, L)]
                cc = u0 * v0 + u1 * v1
                s = jnp.sum(cc)
                return jnp.where(lane == j, s, acc)

            acc = lax.fori_loop(0, L, elem, jnp.zeros((L,), jnp.float32))
            out_v[pl.ds(w0 + g * L, L)] = acc
            return c

        lax.fori_loop(0, W // L, group, 0)
        return carry

    lax.fori_loop(0, BPW // W, window, 0)
    pltpu.sync_copy(out_v, out_hbm.at[pl.ds(base, BPW)])


@jax.jit
def kernel(user, item, user_factors, item_factors):
    k = pl.kernel(
        _body,
        out_type=jax.ShapeDtypeStruct((BATCH,), jnp.float32),
        mesh=plsc.VectorSubcoreMesh(
            core_axis_name="c", subcore_axis_name="s",
            num_cores=NC, num_subcores=NS),
        compiler_params=pltpu.CompilerParams(needs_layout_passes=False),
        scratch_types=[
            pltpu.VMEM((BPW,), jnp.int32),
            pltpu.VMEM((BPW,), jnp.int32),
            pltpu.VMEM((W, NF), jnp.float32),
            pltpu.VMEM((W, NF), jnp.float32),
            pltpu.VMEM((BPW,), jnp.float32),
            pltpu.SemaphoreType.DMA,
            pltpu.SemaphoreType.DMA,
        ],
    )
    return k(user, item, user_factors, item_factors)
